# Initial kernel scaffold; baseline (speedup 1.0000x reference)
import jax
import jax.numpy as jnp
from jax.experimental import pallas as pl


def kernel(sequence, segment_label, token_table, seg_table, pe):
    raise NotImplementedError("write your pallas kernel here")


# speedup vs baseline: 1.2245x; 1.2245x over previous
import functools

import jax
import jax.numpy as jnp
from jax import lax
from jax.experimental import pallas as pl
from jax.experimental.pallas import tpu as pltpu
from jax.experimental.pallas import tpu_sc as plsc

LANES = 16


@functools.lru_cache(maxsize=None)
def _build_sc_gather(N, D, V, C):
    info = plsc.get_sparse_core_info()
    NC, NS = info.num_cores, info.num_subcores
    NW = NC * NS
    assert N % (8 * NW) == 0
    rows_w = N // NW
    CH = 128
    assert rows_w % CH == 0
    nch = rows_w // CH
    L = C // 3

    mesh = plsc.VectorSubcoreMesh(core_axis_name="c", subcore_axis_name="s")

    @functools.partial(
        pl.kernel,
        mesh=mesh,
        compiler_params=pltpu.CompilerParams(use_tc_tiling_on_sc=False),
        out_type=jax.ShapeDtypeStruct((N, D), jnp.float32),
        scratch_types=[
            pltpu.VMEM((rows_w,), jnp.int32),
            pltpu.VMEM((rows_w,), jnp.int32),
            pltpu.VMEM((CH, D), jnp.float32),
            pltpu.VMEM((CH, D), jnp.float32),
            pltpu.SemaphoreType.DMA,
        ],
    )
    def k(seq_hbm, seg_hbm, table_hbm, combo_hbm, out_hbm,
          tokidx_v, cidx_v, tok_v, add_v, sem):
        wid = lax.axis_index("s") * NC + lax.axis_index("c")
        base = wid * rows_w
        pltpu.sync_copy(seq_hbm.at[pl.ds(base, rows_w)], tokidx_v)
        pltpu.sync_copy(seg_hbm.at[pl.ds(base, rows_w)], cidx_v)

        def idx_body(i, _):
            off = i * LANES
            nvec = lax.iota(jnp.int32, LANES) + (base + off)
            lvec = lax.rem(nvec, L)
            cidx_v[pl.ds(off, LANES)] = lvec * 3 + cidx_v[pl.ds(off, LANES)]
            return 0

        lax.fori_loop(0, rows_w // LANES, idx_body, 0)

        def ch_body(j, _):
            off = j * CH
            cp1 = pltpu.async_copy(table_hbm.at[tokidx_v.at[pl.ds(off, CH)]], tok_v, sem)
            cp2 = pltpu.async_copy(combo_hbm.at[cidx_v.at[pl.ds(off, CH)]], add_v, sem)
            cp1.wait()
            cp2.wait()

            def add_body(r, _):
                for c in range(D // LANES):
                    sl = pl.ds(c * LANES, LANES)
                    tok_v[r, sl] = tok_v[r, sl] + add_v[r, sl]
                return 0

            lax.fori_loop(0, CH, add_body, 0)
            pltpu.sync_copy(tok_v, out_hbm.at[pl.ds(base + off, CH)])
            return 0

        lax.fori_loop(0, nch, ch_body, 0)

    return k


def kernel(sequence, segment_label, token_table, seg_table, pe):
    B, L = sequence.shape
    V, D = token_table.shape
    N = B * L
    combo = (pe[0, :L, :][:, None, :] + seg_table[None, :, :]).reshape(L * 3, D)
    seq_flat = sequence.reshape(N).astype(jnp.int32)
    seg_flat = segment_label.reshape(N).astype(jnp.int32)
    k = _build_sc_gather(N, D, V, L * 3)
    out = k(seq_flat, seg_flat, token_table, combo)
    return out.reshape(B, L, D)


# speedup vs baseline: 1.2727x; 1.0393x over previous
import functools

import jax
import jax.numpy as jnp
from jax import lax
from jax.experimental import pallas as pl
from jax.experimental.pallas import tpu as pltpu
from jax.experimental.pallas import tpu_sc as plsc

LANES = 16


@functools.lru_cache(maxsize=None)
def _build_sc_gather(N, D, V, C):
    info = plsc.get_sparse_core_info()
    NC, NS = info.num_cores, info.num_subcores
    NW = NC * NS
    assert N % (8 * NW) == 0
    rows_w = N // NW
    CH = 128
    assert rows_w % (2 * CH) == 0
    nch = rows_w // CH
    L = C // 3

    mesh = plsc.VectorSubcoreMesh(core_axis_name="c", subcore_axis_name="s")

    @functools.partial(
        pl.kernel,
        mesh=mesh,
        compiler_params=pltpu.CompilerParams(use_tc_tiling_on_sc=False),
        out_type=jax.ShapeDtypeStruct((N, D), jnp.float32),
        scratch_types=[
            pltpu.VMEM((rows_w,), jnp.int32),
            pltpu.VMEM((rows_w,), jnp.int32),
            pltpu.VMEM((2, CH, D), jnp.float32),
            pltpu.VMEM((2, CH, D), jnp.float32),
            pltpu.VMEM((2, CH, D), jnp.float32),
            pltpu.SemaphoreType.DMA,
            pltpu.SemaphoreType.DMA,
            pltpu.SemaphoreType.DMA,
            pltpu.SemaphoreType.DMA,
        ],
    )
    def k(seq_hbm, seg_hbm, table_hbm, combo_hbm, out_hbm,
          tokidx_v, cidx_v, tok_v, add_v, out_v, gsa, gsb, wsa, wsb):
        wid = lax.axis_index("s") * NC + lax.axis_index("c")
        base = wid * rows_w
        pltpu.sync_copy(seq_hbm.at[pl.ds(base, rows_w)], tokidx_v)
        pltpu.sync_copy(seg_hbm.at[pl.ds(base, rows_w)], cidx_v)

        def idx_body(i, _):
            for u in range(4):
                off = (i * 4 + u) * LANES
                nvec = lax.iota(jnp.int32, LANES) + (base + off)
                lvec = lax.rem(nvec, L)
                cidx_v[pl.ds(off, LANES)] = lvec * 3 + cidx_v[pl.ds(off, LANES)]
            return 0

        lax.fori_loop(0, rows_w // (4 * LANES), idx_body, 0)

        gsem = (gsa, gsb)
        wsem = (wsa, wsb)

        def fire_gathers(c, p):
            off = c * CH
            pltpu.async_copy(table_hbm.at[tokidx_v.at[pl.ds(off, CH)]],
                             tok_v.at[p], gsem[p])
            pltpu.async_copy(combo_hbm.at[cidx_v.at[pl.ds(off, CH)]],
                             add_v.at[p], gsem[p])

        def wait_gathers(c, p):
            off = c * CH
            pltpu.make_async_copy(table_hbm.at[tokidx_v.at[pl.ds(off, CH)]],
                                  tok_v.at[p], gsem[p]).wait()
            pltpu.make_async_copy(combo_hbm.at[cidx_v.at[pl.ds(off, CH)]],
                                  add_v.at[p], gsem[p]).wait()

        def fire_wb(c, p):
            pltpu.async_copy(out_v.at[p], out_hbm.at[pl.ds(base + c * CH, CH)],
                             wsem[p])

        def wait_wb(c, p):
            pltpu.make_async_copy(out_v.at[p], out_hbm.at[pl.ds(base + c * CH, CH)],
                                  wsem[p]).wait()

        def do_add(p):
            def add_body(r4, _):
                for dr in range(4):
                    r = r4 * 4 + dr
                    for cc in range(D // LANES):
                        sl = pl.ds(cc * LANES, LANES)
                        out_v[p, r, sl] = tok_v[p, r, sl] + add_v[p, r, sl]
                return 0

            lax.fori_loop(0, CH // 4, add_body, 0)

        fire_gathers(0, 0)
        fire_gathers(1, 1)

        def pipe_body(i, _):
            for p in range(2):
                c = i * 2 + p
                wait_gathers(c, p)

                @pl.when(i > 0)
                def _():
                    wait_wb(c - 2, p)

                do_add(p)
                fire_wb(c, p)

                @pl.when(c + 2 < nch)
                def _():
                    fire_gathers(c + 2, p)

            return 0

        lax.fori_loop(0, nch // 2, pipe_body, 0)
        wait_wb(nch - 2, 0)
        wait_wb(nch - 1, 1)

    return k


def kernel(sequence, segment_label, token_table, seg_table, pe):
    B, L = sequence.shape
    V, D = token_table.shape
    N = B * L
    combo = (pe[0, :L, :][:, None, :] + seg_table[None, :, :]).reshape(L * 3, D)
    seq_flat = sequence.reshape(N).astype(jnp.int32)
    seg_flat = segment_label.reshape(N).astype(jnp.int32)
    k = _build_sc_gather(N, D, V, L * 3)
    out = k(seq_flat, seg_flat, token_table, combo)
    return out.reshape(B, L, D)


# speedup vs baseline: 1.2760x; 1.0026x over previous
import functools

import jax
import jax.numpy as jnp
from jax import lax
from jax.experimental import pallas as pl
from jax.experimental.pallas import tpu as pltpu
from jax.experimental.pallas import tpu_sc as plsc

LANES = 16


@functools.lru_cache(maxsize=None)
def _build_sc_gather(N, D, V, C):
    info = plsc.get_sparse_core_info()
    NC, NS = info.num_cores, info.num_subcores
    NW = NC * NS
    assert N % (8 * NW) == 0
    rows_w = N // NW
    CH = 128
    assert rows_w % (2 * CH) == 0
    nch = rows_w // CH
    L = C // 3

    mesh = plsc.VectorSubcoreMesh(core_axis_name="c", subcore_axis_name="s")

    @functools.partial(
        pl.kernel,
        mesh=mesh,
        compiler_params=pltpu.CompilerParams(use_tc_tiling_on_sc=False),
        out_type=jax.ShapeDtypeStruct((N, D), jnp.float32),
        scratch_types=[
            pltpu.VMEM((rows_w,), jnp.int32),
            pltpu.VMEM((rows_w,), jnp.int32),
            pltpu.VMEM((2, CH, D), jnp.float32),
            pltpu.VMEM((2, CH, D), jnp.float32),
            pltpu.VMEM((2, CH, D), jnp.float32),
            pltpu.SemaphoreType.DMA,
            pltpu.SemaphoreType.DMA,
            pltpu.SemaphoreType.DMA,
            pltpu.SemaphoreType.DMA,
        ],
    )
    def k(seq_hbm, seg_hbm, table_hbm, combo_hbm, out_hbm,
          tokidx_v, cidx_v, tok_v, add_v, out_v, gsa, gsb, wsa, wsb):
        wid = lax.axis_index("s") * NC + lax.axis_index("c")
        base = wid * rows_w
        pltpu.sync_copy(seq_hbm.at[pl.ds(base, rows_w)], tokidx_v)
        pltpu.sync_copy(seg_hbm.at[pl.ds(base, rows_w)], cidx_v)

        def idx_body(i, _):
            for u in range(4):
                off = (i * 4 + u) * LANES
                nvec = lax.iota(jnp.int32, LANES) + (base + off)
                lvec = lax.rem(nvec, L)
                cidx_v[pl.ds(off, LANES)] = lvec * 3 + cidx_v[pl.ds(off, LANES)]
            return 0

        lax.fori_loop(0, rows_w // (4 * LANES), idx_body, 0)

        gsem = (gsa, gsb)
        wsem = (wsa, wsb)

        def fire_gathers(c, p):
            off = c * CH
            pltpu.async_copy(table_hbm.at[tokidx_v.at[pl.ds(off, CH)]],
                             tok_v.at[p], gsem[p])
            pltpu.async_copy(combo_hbm.at[cidx_v.at[pl.ds(off, CH)]],
                             add_v.at[p], gsem[p])

        def wait_gathers(c, p):
            off = c * CH
            pltpu.make_async_copy(table_hbm.at[tokidx_v.at[pl.ds(off, CH)]],
                                  tok_v.at[p], gsem[p]).wait()
            pltpu.make_async_copy(combo_hbm.at[cidx_v.at[pl.ds(off, CH)]],
                                  add_v.at[p], gsem[p]).wait()

        def fire_wb(c, p):
            pltpu.async_copy(out_v.at[p], out_hbm.at[pl.ds(base + c * CH, CH)],
                             wsem[p])

        def wait_wb(c, p):
            pltpu.make_async_copy(out_v.at[p], out_hbm.at[pl.ds(base + c * CH, CH)],
                                  wsem[p]).wait()

        def do_add(p):
            def add_body(r4, _):
                for dr in range(4):
                    r = r4 * 4 + dr
                    for cc in range(D // LANES):
                        sl = pl.ds(cc * LANES, LANES)
                        out_v[p, r, sl] = tok_v[p, r, sl] + add_v[p, r, sl]
                return 0

            lax.fori_loop(0, CH // 4, add_body, 0)

        fire_gathers(0, 0)
        fire_gathers(1, 1)

        def pipe_body(i, _):
            for p in range(2):
                c = i * 2 + p
                wait_gathers(c, p)

                @pl.when(i > 0)
                def _():
                    wait_wb(c - 2, p)

                do_add(p)
                fire_wb(c, p)

                @pl.when(c + 2 < nch)
                def _():
                    fire_gathers(c + 2, p)

            return 0

        lax.fori_loop(0, nch // 2, pipe_body, 0)
        wait_wb(nch - 2, 0)
        wait_wb(nch - 1, 1)

    return k


def kernel(sequence, segment_label, token_table, seg_table, pe):
    B, L = sequence.shape
    V, D = token_table.shape
    N = B * L
    combo = (pe[0, :L, :][:, None, :] + seg_table[None, :, :]).reshape(L * 3, D)
    seq_flat = sequence.reshape(N).astype(jnp.int32)
    seg_flat = segment_label.reshape(N).astype(jnp.int32)
    one = (1 - segment_label[0, 0] * 0).astype(token_table.dtype)
    table_rm = token_table * one
    k = _build_sc_gather(N, D, V, L * 3)
    out = k(seq_flat, seg_flat, table_rm, combo)
    return out.reshape(B, L, D)
